# Initial kernel scaffold; baseline (speedup 1.0000x reference)
import jax
import jax.numpy as jnp
from jax import lax
from jax.experimental import pallas as pl
from jax.experimental.pallas import tpu as pltpu
from jax.experimental.pallas import tpu_sc as plsc

DIM = 32
B_TOTAL = 16384 * 50

NC = 2
NS = 16
NW = NC * NS
B_PER_W = B_TOTAL // NW
CHUNK = 400
N_CHUNKS = B_PER_W // CHUNK
CONV_UNROLL = 4
LANES = 16


def _body(x_hbm, table_hbm, out_hbm, x_v, idx_v, buf0, buf1, sem0, sem1):
    wid = lax.axis_index("s") * NC + lax.axis_index("c")
    base = wid * B_PER_W

    pltpu.sync_copy(x_hbm.at[pl.ds(base, B_PER_W)], x_v)

    def conv(i, carry):
        for u in range(CONV_UNROLL):
            off = i * (LANES * CONV_UNROLL) + u * LANES
            v = x_v[pl.ds(off, LANES)] + 1.0
            v = jnp.where(v != v, 0.0, v)
            idx_v[pl.ds(off, LANES)] = v.astype(jnp.int32)
        return carry

    lax.fori_loop(0, B_PER_W // (LANES * CONV_UNROLL), conv, 0)

    bufs = (buf0, buf1)
    sems = (sem0, sem1)

    def gather(c, b):
        pltpu.make_async_copy(
            table_hbm.at[idx_v.at[pl.ds(c * CHUNK, CHUNK)]], bufs[b], sems[b]
        ).start()

    def gwait(b):
        pltpu.make_async_copy(
            table_hbm.at[idx_v.at[pl.ds(0, CHUNK)]], bufs[b], sems[b]
        ).wait()

    def scatter(c, b):
        pltpu.sync_copy(bufs[b], out_hbm.at[pl.ds(base + c * CHUNK, CHUNK)])

    gather(0, 0)

    def step(i, carry):
        c0 = 2 * i
        gather(c0 + 1, 1)
        gwait(0)
        scatter(c0, 0)
        gather(c0 + 2, 0)
        gwait(1)
        scatter(c0 + 1, 1)
        return carry

    lax.fori_loop(0, N_CHUNKS // 2 - 1, step, 0)

    c0 = N_CHUNKS - 2
    gather(c0 + 1, 1)
    gwait(0)
    scatter(c0, 0)
    gwait(1)
    scatter(c0 + 1, 1)


@jax.jit
def kernel(x, table):
    mesh = plsc.VectorSubcoreMesh(core_axis_name="c", subcore_axis_name="s")
    xf = x.reshape(B_TOTAL)
    out = pl.kernel(
        _body,
        mesh=mesh,
        out_type=jax.ShapeDtypeStruct((B_TOTAL, DIM), jnp.float32),
        scratch_types=[
            pltpu.VMEM((B_PER_W,), jnp.float32),
            pltpu.VMEM((B_PER_W,), jnp.int32),
            pltpu.VMEM((CHUNK, DIM), jnp.float32),
            pltpu.VMEM((CHUNK, DIM), jnp.float32),
            pltpu.SemaphoreType.DMA,
            pltpu.SemaphoreType.DMA,
        ],
    )(xf, table)
    return out.reshape(x.shape[0], x.shape[1], DIM)


# speedup vs baseline: 3.0009x; 3.0009x over previous
import jax
import jax.numpy as jnp
from jax import lax
from jax.experimental import pallas as pl
from jax.experimental.pallas import tpu as pltpu
from jax.experimental.pallas import tpu_sc as plsc

DIM = 32
B_TOTAL = 16384 * 50

NC = 2
NS = 16
NW = NC * NS
B_PER_W = B_TOTAL // NW
CHUNK = 400
N_CHUNKS = B_PER_W // CHUNK
CONV_UNROLL = 4
LANES = 16


def _body(x_hbm, table_hbm, out_hbm, x_v, idx_v, buf0, buf1, sem0, sem1):
    wid = lax.axis_index("s") * NC + lax.axis_index("c")
    base = wid * B_PER_W

    pltpu.sync_copy(x_hbm.at[pl.ds(base, B_PER_W)], x_v)

    def conv(i, carry):
        for u in range(CONV_UNROLL):
            off = i * (LANES * CONV_UNROLL) + u * LANES
            v = x_v[pl.ds(off, LANES)] + 1.0
            v = jnp.where(v != v, 0.0, v)
            idx_v[pl.ds(off, LANES)] = v.astype(jnp.int32)
        return carry

    lax.fori_loop(0, B_PER_W // (LANES * CONV_UNROLL), conv, 0)

    bufs = (buf0, buf1)
    sems = (sem0, sem1)

    def gather(c, b):
        pltpu.make_async_copy(
            table_hbm.at[idx_v.at[pl.ds(c * CHUNK, CHUNK)]], bufs[b], sems[b]
        ).start()

    def gwait(b):
        pltpu.make_async_copy(
            table_hbm.at[idx_v.at[pl.ds(0, CHUNK)]], bufs[b], sems[b]
        ).wait()

    def scatter(c, b):
        pltpu.sync_copy(bufs[b], out_hbm.at[pl.ds(base + c * CHUNK, CHUNK)])

    gather(0, 0)

    def step(i, carry):
        c0 = 2 * i
        gather(c0 + 1, 1)
        gwait(0)
        scatter(c0, 0)
        gather(c0 + 2, 0)
        gwait(1)
        scatter(c0 + 1, 1)
        return carry

    lax.fori_loop(0, N_CHUNKS // 2 - 1, step, 0)

    c0 = N_CHUNKS - 2
    gather(c0 + 1, 1)
    gwait(0)
    scatter(c0, 0)
    gwait(1)
    scatter(c0 + 1, 1)


@jax.jit
def kernel(x, table):
    mesh = plsc.VectorSubcoreMesh(core_axis_name="c", subcore_axis_name="s")
    xf = x.reshape(B_TOTAL)
    out = pl.kernel(
        _body,
        mesh=mesh,
        out_type=jax.ShapeDtypeStruct((B_TOTAL, DIM), jnp.float32),
        scratch_types=[
            pltpu.VMEM((B_PER_W,), jnp.float32),
            pltpu.VMEM((B_PER_W,), jnp.int32),
            pltpu.VMEM((CHUNK, DIM), jnp.float32),
            pltpu.VMEM((CHUNK, DIM), jnp.float32),
            pltpu.SemaphoreType.DMA,
            pltpu.SemaphoreType.DMA,
        ],
        compiler_params=pltpu.CompilerParams(use_tc_tiling_on_sc=False),
    )(xf, table)
    return out.reshape(x.shape[0], x.shape[1], DIM)


# speedup vs baseline: 3.0032x; 1.0007x over previous
import jax
import jax.numpy as jnp
from jax import lax
from jax.experimental import pallas as pl
from jax.experimental.pallas import tpu as pltpu
from jax.experimental.pallas import tpu_sc as plsc

DIM = 32
B_TOTAL = 16384 * 50

NC = 2
NS = 16
NW = NC * NS
B_PER_W = B_TOTAL // NW
CHUNK = 400
N_CHUNKS = B_PER_W // CHUNK
CONV_UNROLL = 4
LANES = 16


NBUF = 4


def _body(x_hbm, table_hbm, out_hbm, x_v, idx_v, bufs, gsems, ssems):
    wid = lax.axis_index("s") * NC + lax.axis_index("c")
    base = wid * B_PER_W

    pltpu.sync_copy(x_hbm.at[pl.ds(base, B_PER_W)], x_v)

    def conv_all():
        def body(i, carry):
            for u in range(CONV_UNROLL):
                off = i * (LANES * CONV_UNROLL) + u * LANES
                v = x_v[pl.ds(off, LANES)] + 1.0
                v = jnp.where(v != v, 0.0, v)
                idx_v[pl.ds(off, LANES)] = v.astype(jnp.int32)
            return carry

        lax.fori_loop(0, B_PER_W // (LANES * CONV_UNROLL), body, 0)

    def gstart(c, b):
        pltpu.make_async_copy(
            table_hbm.at[idx_v.at[pl.ds(c * CHUNK, CHUNK)]], bufs[b], gsems[b]
        ).start()

    def gwait(b):
        pltpu.make_async_copy(
            table_hbm.at[idx_v.at[pl.ds(0, CHUNK)]], bufs[b], gsems[b]
        ).wait()

    def sstart(c, b):
        pltpu.make_async_copy(
            bufs[b], out_hbm.at[pl.ds(base + c * CHUNK, CHUNK)], ssems[b]
        ).start()

    def swait(b):
        pltpu.make_async_copy(
            bufs[b], out_hbm.at[pl.ds(base, CHUNK)], ssems[b]
        ).wait()

    conv_all()

    gstart(0, 0)
    gstart(1, 1)
    gwait(0)
    sstart(0, 0)
    gstart(2, 2)
    gwait(1)
    sstart(1, 1)
    gstart(3, 3)

    def step(i, carry):
        for k in range(NBUF):
            j = 2 + i * NBUF + k
            b = (2 + k) % NBUF
            gwait(b)
            sstart(j, b)
            swait((b + 2) % NBUF)
            gstart(j + 2, (b + 2) % NBUF)
        return carry

    lax.fori_loop(0, (N_CHUNKS - 4) // NBUF, step, 0)

    j = N_CHUNKS - 2
    gwait(j % NBUF)
    sstart(j, j % NBUF)
    gwait((j + 1) % NBUF)
    sstart(j + 1, (j + 1) % NBUF)
    for b in range(NBUF):
        swait(b)


@jax.jit
def kernel(x, table):
    mesh = plsc.VectorSubcoreMesh(core_axis_name="c", subcore_axis_name="s")
    xf = x.reshape(B_TOTAL)
    out = pl.kernel(
        _body,
        mesh=mesh,
        out_type=jax.ShapeDtypeStruct((B_TOTAL, DIM), jnp.float32),
        scratch_types=[
            pltpu.VMEM((B_PER_W,), jnp.float32),
            pltpu.VMEM((B_PER_W,), jnp.int32),
            tuple(pltpu.VMEM((CHUNK, DIM), jnp.float32) for _ in range(NBUF)),
            tuple(pltpu.SemaphoreType.DMA for _ in range(NBUF)),
            tuple(pltpu.SemaphoreType.DMA for _ in range(NBUF)),
        ],
        compiler_params=pltpu.CompilerParams(use_tc_tiling_on_sc=False),
    )(xf, table)
    return out.reshape(x.shape[0], x.shape[1], DIM)


# speedup vs baseline: 5.0508x; 1.6818x over previous
import jax
import jax.numpy as jnp
from jax import lax
from jax.experimental import pallas as pl
from jax.experimental.pallas import tpu as pltpu
from jax.experimental.pallas import tpu_sc as plsc

DIM = 32
B_TOTAL = 16384 * 50

NC = 2
NS = 16
NW = NC * NS
LANES = 16

A_PER_W = 4
HC = 5
CHUNK = HC * 128
N_CHUNKS = 50 // HC
BLK_W = 128 * 50


def _body(x_hbm, table_hbm, out_hbm, x_v, idx_v, gb0, gb1, tb0, tb1,
          gsem0, gsem1, ssem0, ssem1):
    wid = lax.axis_index("s") * NC + lax.axis_index("c")

    iota = lax.iota(jnp.int32, LANES)
    iota50 = iota * 50
    gbufs = (gb0, gb1)
    tbufs = (tb0, tb1)
    gsems = (gsem0, gsem1)
    ssems = (ssem0, ssem1)

    def gstart(a, cc, b):
        pltpu.make_async_copy(
            table_hbm.at[idx_v.at[pl.ds(cc * CHUNK, CHUNK)]],
            gbufs[b], gsems[b],
        ).start()

    def gwait(b):
        pltpu.make_async_copy(
            table_hbm.at[idx_v.at[pl.ds(0, CHUNK)]], gbufs[b], gsems[b]
        ).wait()

    def sstart(a, cc, b):
        for hh in range(HC):
            h = cc * HC + hh
            for e in range(DIM // 8):
                pltpu.make_async_copy(
                    tbufs[b].at[hh, e], out_hbm.at[h, e, a], ssems[b]
                ).start()

    def swait(b):
        for _ in range(HC * (DIM // 8)):
            pltpu.make_async_copy(
                tbufs[b].at[0, 0], out_hbm.at[0, 0, 0], ssems[b]
            ).wait()

    def conv(blk, carry):
        def body(k, carry2):
            h = k // 8
            c0 = (k % 8) * LANES
            src = iota50 + (c0 * 50 + h)
            v = plsc.load_gather(x_v, [src]) + 1.0
            v = jnp.where(v != v, 0.0, v)
            idx_v[pl.ds(k * LANES, LANES)] = v.astype(jnp.int32)
            return carry2

        return lax.fori_loop(0, BLK_W // LANES, body, carry)

    def transpose(b):
        gb, tb = gbufs[b], tbufs[b]
        for hh in range(HC):
            for e in range(DIM // 8):
                def tbody(cv, carry, hh=hh, e=e):
                    row = iota + (hh * 128 + cv * LANES)
                    for f in range(8):
                        col = jnp.full((LANES,), 8 * e + f, jnp.int32)
                        tb[hh, e, f, pl.ds(cv * LANES, LANES)] = (
                            plsc.load_gather(gb, [row, col])
                        )
                    return carry

                lax.fori_loop(0, 128 // LANES, tbody, 0)

    def chunk_step(a, cc, b):
        gwait(b)
        swait(b)
        transpose(b)
        sstart(a, cc, b)

    a_last = wid * A_PER_W + (A_PER_W - 1)
    sstart(a_last, N_CHUNKS - 1, 0)
    sstart(a_last, N_CHUNKS - 1, 1)

    def ablock(blk, carry):
        a = wid * A_PER_W + blk
        pltpu.sync_copy(x_hbm.at[pl.ds(a * BLK_W, BLK_W)], x_v)
        conv(blk, 0)
        gstart(a, 0, 0)
        gstart(a, 1, 1)

        def pair(j, carry2):
            c0 = 2 * j
            chunk_step(a, c0, 0)
            gstart(a, c0 + 2, 0)
            chunk_step(a, c0 + 1, 1)
            gstart(a, c0 + 3, 1)
            return carry2

        lax.fori_loop(0, (N_CHUNKS - 2) // 2, pair, 0)
        chunk_step(a, N_CHUNKS - 2, 0)
        chunk_step(a, N_CHUNKS - 1, 1)
        return carry

    lax.fori_loop(0, A_PER_W, ablock, 0)
    swait(0)
    swait(1)


def kernel(x, table):
    mesh = plsc.VectorSubcoreMesh(core_axis_name="c", subcore_axis_name="s")
    xf = x.reshape(B_TOTAL)
    out = pl.kernel(
        _body,
        mesh=mesh,
        out_type=jax.ShapeDtypeStruct((50, DIM // 8, 128, 8, 128),
                                      jnp.float32),
        scratch_types=[
            pltpu.VMEM((BLK_W,), jnp.float32),
            pltpu.VMEM((BLK_W,), jnp.int32),
            pltpu.VMEM((CHUNK, DIM), jnp.float32),
            pltpu.VMEM((CHUNK, DIM), jnp.float32),
            pltpu.VMEM((HC, DIM // 8, 8, 128), jnp.float32),
            pltpu.VMEM((HC, DIM // 8, 8, 128), jnp.float32),
            pltpu.SemaphoreType.DMA,
            pltpu.SemaphoreType.DMA,
            pltpu.SemaphoreType.DMA,
            pltpu.SemaphoreType.DMA,
        ],
        compiler_params=pltpu.CompilerParams(
            use_tc_tiling_on_sc=False, needs_layout_passes=False
        ),
    )(xf, table)
    return jnp.transpose(out, (2, 4, 0, 1, 3)).reshape(16384, 50, DIM)


# speedup vs baseline: 6.3006x; 1.2475x over previous
import jax
import jax.numpy as jnp
from jax import lax
from jax.experimental import pallas as pl
from jax.experimental.pallas import tpu as pltpu
from jax.experimental.pallas import tpu_sc as plsc

DIM = 32
B_TOTAL = 16384 * 50

NC = 2
NS = 16
NW = NC * NS
LANES = 16

A_PER_W = 4
HC = 5
CHUNK = HC * 128
N_CHUNKS = 50 // HC
BLK_W = 128 * 50


def _body(x_hbm, table_hbm, out_hbm, x_v, idx_v, gb0, gb1, tb0, tb1,
          gsem0, gsem1, ssem0, ssem1):
    wid = lax.axis_index("s") * NC + lax.axis_index("c")

    iota = lax.iota(jnp.int32, LANES)
    iota50 = iota * 50
    gbufs = (gb0, gb1)
    tbufs = (tb0, tb1)
    gsems = (gsem0, gsem1)
    ssems = (ssem0, ssem1)

    def gstart(a, cc, b):
        pltpu.make_async_copy(
            table_hbm.at[idx_v.at[pl.ds(cc * CHUNK, CHUNK)]],
            gbufs[b], gsems[b],
        ).start()

    def gwait(b):
        pltpu.make_async_copy(
            table_hbm.at[idx_v.at[pl.ds(0, CHUNK)]], gbufs[b], gsems[b]
        ).wait()

    def sstart(a, cc, b):
        for hh in range(HC):
            h = cc * HC + hh
            for e in range(DIM // 8):
                pltpu.make_async_copy(
                    tbufs[b].at[hh, pl.ds(e * 1024, 1024)],
                    out_hbm.at[h, e, a], ssems[b]
                ).start()

    def swait(b):
        for _ in range(HC * (DIM // 8)):
            pltpu.make_async_copy(
                tbufs[b].at[0, pl.ds(0, 1024)], out_hbm.at[0, 0, 0],
                ssems[b]
            ).wait()

    def conv(blk, carry):
        def body(k, carry2):
            h = k // 8
            c0 = (k % 8) * LANES
            src = iota50 + (c0 * 50 + h)
            v = plsc.load_gather(x_v, [src]) + 1.0
            v = jnp.where(v != v, 0.0, v)
            idx_v[pl.ds(k * LANES, LANES)] = v.astype(jnp.int32)
            return carry2

        return lax.fori_loop(0, BLK_W // LANES, body, carry)

    iota128 = iota * 128

    def transpose(b):
        gb, tb = gbufs[b], tbufs[b]
        for hh in range(HC):
            def tbody(c, carry, hh=hh):
                r = hh * 128 + c
                v0 = gb[r, pl.ds(0, LANES)]
                v1 = gb[r, pl.ds(LANES, LANES)]
                i0 = iota128 + c
                plsc.store_scatter(tb.at[hh], [i0], v0)
                plsc.store_scatter(tb.at[hh], [i0 + 2048], v1)
                return carry

            lax.fori_loop(0, 128, tbody, 0)

    def chunk_step(a, cc, b):
        gwait(b)
        swait(b)
        transpose(b)
        sstart(a, cc, b)

    a_last = wid * A_PER_W + (A_PER_W - 1)
    sstart(a_last, N_CHUNKS - 1, 0)
    sstart(a_last, N_CHUNKS - 1, 1)

    def ablock(blk, carry):
        a = wid * A_PER_W + blk
        pltpu.sync_copy(x_hbm.at[pl.ds(a * BLK_W, BLK_W)], x_v)
        conv(blk, 0)
        gstart(a, 0, 0)
        gstart(a, 1, 1)

        def pair(j, carry2):
            c0 = 2 * j
            chunk_step(a, c0, 0)
            gstart(a, c0 + 2, 0)
            chunk_step(a, c0 + 1, 1)
            gstart(a, c0 + 3, 1)
            return carry2

        lax.fori_loop(0, (N_CHUNKS - 2) // 2, pair, 0)
        chunk_step(a, N_CHUNKS - 2, 0)
        chunk_step(a, N_CHUNKS - 1, 1)
        return carry

    lax.fori_loop(0, A_PER_W, ablock, 0)
    swait(0)
    swait(1)


def kernel(x, table):
    mesh = plsc.VectorSubcoreMesh(core_axis_name="c", subcore_axis_name="s")
    xf = x.reshape(B_TOTAL)
    out = pl.kernel(
        _body,
        mesh=mesh,
        out_type=jax.ShapeDtypeStruct((50, DIM // 8, 128, 8 * 128),
                                      jnp.float32),
        scratch_types=[
            pltpu.VMEM((BLK_W,), jnp.float32),
            pltpu.VMEM((BLK_W,), jnp.int32),
            pltpu.VMEM((CHUNK, DIM), jnp.float32),
            pltpu.VMEM((CHUNK, DIM), jnp.float32),
            pltpu.VMEM((HC, DIM * 128), jnp.float32),
            pltpu.VMEM((HC, DIM * 128), jnp.float32),
            pltpu.SemaphoreType.DMA,
            pltpu.SemaphoreType.DMA,
            pltpu.SemaphoreType.DMA,
            pltpu.SemaphoreType.DMA,
        ],
        compiler_params=pltpu.CompilerParams(
            use_tc_tiling_on_sc=False, needs_layout_passes=False
        ),
    )(xf, table)
    out5 = out.reshape(50, DIM // 8, 128, 8, 128)
    return jnp.transpose(out5, (2, 4, 0, 1, 3)).reshape(16384, 50, DIM)


# speedup vs baseline: 7.5486x; 1.1981x over previous
import jax
import jax.numpy as jnp
from jax import lax
from jax.experimental import pallas as pl
from jax.experimental.pallas import tpu as pltpu
from jax.experimental.pallas import tpu_sc as plsc

DIM = 32
B_TOTAL = 16384 * 50

NC = 2
NS = 16
NW = NC * NS
LANES = 16

A_PER_W = 4
HC = 5
CHUNK = HC * 128
N_CHUNKS = 50 // HC
BLK_W = 128 * 50


def _body(x_hbm, table_hbm, out_hbm, x_v, idx_v, gb0, gb1, tb0, tb1,
          gsem0, gsem1, ssem0, ssem1):
    wid = lax.axis_index("s") * NC + lax.axis_index("c")

    iota = lax.iota(jnp.int32, LANES)
    iota50 = iota * 50
    gbufs = (gb0, gb1)
    tbufs = (tb0, tb1)
    gsems = (gsem0, gsem1)
    ssems = (ssem0, ssem1)

    def gstart(a, cc, b):
        pltpu.make_async_copy(
            table_hbm.at[idx_v.at[pl.ds(cc * CHUNK, CHUNK)]],
            gbufs[b], gsems[b],
        ).start()

    def gwait(b):
        pltpu.make_async_copy(
            table_hbm.at[idx_v.at[pl.ds(0, CHUNK)]], gbufs[b], gsems[b]
        ).wait()

    def sstart(a, cc, b):
        for hh in range(HC):
            h = cc * HC + hh
            for e in range(DIM // 8):
                pltpu.make_async_copy(
                    tbufs[b].at[hh, pl.ds(e * 1024, 1024)],
                    out_hbm.at[h, e, a], ssems[b]
                ).start()

    def swait(b):
        for _ in range(HC * (DIM // 8)):
            pltpu.make_async_copy(
                tbufs[b].at[0, pl.ds(0, 1024)], out_hbm.at[0, 0, 0],
                ssems[b]
            ).wait()

    def conv(blk):
        @plsc.parallel_loop(0, BLK_W // LANES, unroll=4)
        def body(k):
            h = k // 8
            c0 = (k % 8) * LANES
            src = iota50 + (c0 * 50 + h)
            v = plsc.load_gather(x_v, [src]) + 1.0
            v = jnp.where(v != v, 0.0, v)
            idx_v[pl.ds(k * LANES, LANES)] = v.astype(jnp.int32)

    iota128 = iota * 128

    def transpose(b):
        gb, tb = gbufs[b], tbufs[b]
        for hh in range(HC):
            @plsc.parallel_loop(0, 128, unroll=4)
            def tbody(c, hh=hh):
                r = hh * 128 + c
                v0 = gb[r, pl.ds(0, LANES)]
                v1 = gb[r, pl.ds(LANES, LANES)]
                i0 = iota128 + c
                plsc.store_scatter(tb.at[hh], [i0], v0)
                plsc.store_scatter(tb.at[hh], [i0 + 2048], v1)

    def chunk_step(a, cc, b):
        gwait(b)
        swait(b)
        transpose(b)
        sstart(a, cc, b)

    a_last = wid * A_PER_W + (A_PER_W - 1)
    sstart(a_last, N_CHUNKS - 1, 0)
    sstart(a_last, N_CHUNKS - 1, 1)

    def ablock(blk, carry):
        a = wid * A_PER_W + blk
        pltpu.sync_copy(x_hbm.at[pl.ds(a * BLK_W, BLK_W)], x_v)
        conv(blk)
        gstart(a, 0, 0)
        gstart(a, 1, 1)

        def pair(j, carry2):
            c0 = 2 * j
            chunk_step(a, c0, 0)
            gstart(a, c0 + 2, 0)
            chunk_step(a, c0 + 1, 1)
            gstart(a, c0 + 3, 1)
            return carry2

        lax.fori_loop(0, (N_CHUNKS - 2) // 2, pair, 0)
        chunk_step(a, N_CHUNKS - 2, 0)
        chunk_step(a, N_CHUNKS - 1, 1)
        return carry

    lax.fori_loop(0, A_PER_W, ablock, 0)
    swait(0)
    swait(1)


def kernel(x, table):
    mesh = plsc.VectorSubcoreMesh(core_axis_name="c", subcore_axis_name="s")
    xf = x.reshape(B_TOTAL)
    out = pl.kernel(
        _body,
        mesh=mesh,
        out_type=jax.ShapeDtypeStruct((50, DIM // 8, 128, 8 * 128),
                                      jnp.float32),
        scratch_types=[
            pltpu.VMEM((BLK_W,), jnp.float32),
            pltpu.VMEM((BLK_W,), jnp.int32),
            pltpu.VMEM((CHUNK, DIM), jnp.float32),
            pltpu.VMEM((CHUNK, DIM), jnp.float32),
            pltpu.VMEM((HC, DIM * 128), jnp.float32),
            pltpu.VMEM((HC, DIM * 128), jnp.float32),
            pltpu.SemaphoreType.DMA,
            pltpu.SemaphoreType.DMA,
            pltpu.SemaphoreType.DMA,
            pltpu.SemaphoreType.DMA,
        ],
        compiler_params=pltpu.CompilerParams(
            use_tc_tiling_on_sc=False, needs_layout_passes=False
        ),
    )(xf, table)
    out5 = out.reshape(50, DIM // 8, 128, 8, 128)
    return jnp.transpose(out5, (2, 4, 0, 1, 3)).reshape(16384, 50, DIM)


# speedup vs baseline: 21.8008x; 2.8881x over previous
import jax
import jax.numpy as jnp
from jax import lax
from jax.experimental import pallas as pl
from jax.experimental.pallas import tpu as pltpu
from jax.experimental.pallas import tpu_sc as plsc

DIM = 32
B_TOTAL = 16384 * 50

NC = 2
NS = 16
NW = NC * NS
LANES = 16

A_PER_W = 4
HC = 5
CHUNK = HC * 128
N_CHUNKS = 50 // HC
BLK_W = 128 * 50


def _body(x_hbm, table_hbm, out_hbm, x_v, idx_v, gb0, gb1, tb0, tb1,
          gsem0, gsem1, ssem0, ssem1):
    wid = lax.axis_index("s") * NC + lax.axis_index("c")

    iota = lax.iota(jnp.int32, LANES)
    iota50 = iota * 50
    gbufs = (gb0, gb1)
    tbufs = (tb0, tb1)
    gsems = (gsem0, gsem1)
    ssems = (ssem0, ssem1)

    def gstart(a, cc, b):
        pltpu.make_async_copy(
            table_hbm.at[idx_v.at[pl.ds(cc * CHUNK, CHUNK)]],
            gbufs[b], gsems[b],
        ).start()

    def gwait(b):
        pltpu.make_async_copy(
            table_hbm.at[idx_v.at[pl.ds(0, CHUNK)]], gbufs[b], gsems[b]
        ).wait()

    def sstart(a, cc, b):
        for hh in range(HC):
            h = cc * HC + hh
            for e in range(DIM // 8):
                pltpu.make_async_copy(
                    tbufs[b].at[hh, pl.ds(e * 8, 8), pl.ds(0, 128)],
                    out_hbm.at[h, e, a], ssems[b]
                ).start()

    def swait(b):
        for _ in range(HC * (DIM // 8)):
            pltpu.make_async_copy(
                tbufs[b].at[0, pl.ds(0, 8), pl.ds(0, 128)],
                out_hbm.at[0, 0, 0], ssems[b]
            ).wait()

    def conv(blk):
        @plsc.parallel_loop(0, BLK_W // LANES, unroll=4)
        def body(k):
            h = k // 8
            c0 = (k % 8) * LANES
            src = iota50 + (c0 * 50 + h)
            v = plsc.load_gather(x_v, [src]) + 1.0
            v = jnp.where(v != v, 0.0, v)
            idx_v[pl.ds(k * LANES, LANES)] = v.astype(jnp.int32)

    def transpose(b):
        gb, tb = gbufs[b], tbufs[b]
        for hh in range(HC):
            @plsc.parallel_loop(0, 128, unroll=4)
            def tbody(c, hh=hh):
                r = hh * 128 + c
                v0 = gb[r, pl.ds(0, LANES)]
                v1 = gb[r, pl.ds(LANES, LANES)]
                cb = jnp.full((LANES,), c, jnp.int32)
                plsc.store_scatter(tb.at[hh], [iota, cb], v0)
                plsc.store_scatter(tb.at[hh], [iota + LANES, cb], v1)

    def chunk_step(a, cc, b):
        gwait(b)
        swait(b)
        transpose(b)
        sstart(a, cc, b)

    a_last = wid * A_PER_W + (A_PER_W - 1)
    sstart(a_last, N_CHUNKS - 1, 0)
    sstart(a_last, N_CHUNKS - 1, 1)

    def ablock(blk, carry):
        a = wid * A_PER_W + blk
        pltpu.sync_copy(x_hbm.at[pl.ds(a * BLK_W, BLK_W)], x_v)
        conv(blk)
        gstart(a, 0, 0)
        gstart(a, 1, 1)

        def pair(j, carry2):
            c0 = 2 * j
            chunk_step(a, c0, 0)
            gstart(a, c0 + 2, 0)
            chunk_step(a, c0 + 1, 1)
            gstart(a, c0 + 3, 1)
            return carry2

        lax.fori_loop(0, (N_CHUNKS - 2) // 2, pair, 0)
        chunk_step(a, N_CHUNKS - 2, 0)
        chunk_step(a, N_CHUNKS - 1, 1)
        return carry

    lax.fori_loop(0, A_PER_W, ablock, 0)
    swait(0)
    swait(1)


def kernel(x, table):
    mesh = plsc.VectorSubcoreMesh(core_axis_name="c", subcore_axis_name="s")
    xf = x.reshape(B_TOTAL)
    out = pl.kernel(
        _body,
        mesh=mesh,
        out_type=jax.ShapeDtypeStruct((50, DIM // 8, 128, 8, 128),
                                      jnp.float32),
        scratch_types=[
            pltpu.VMEM((BLK_W,), jnp.float32),
            pltpu.VMEM((BLK_W,), jnp.int32),
            pltpu.VMEM((CHUNK, DIM), jnp.float32),
            pltpu.VMEM((CHUNK, DIM), jnp.float32),
            pltpu.VMEM((HC, DIM, 129), jnp.float32),
            pltpu.VMEM((HC, DIM, 129), jnp.float32),
            pltpu.SemaphoreType.DMA,
            pltpu.SemaphoreType.DMA,
            pltpu.SemaphoreType.DMA,
            pltpu.SemaphoreType.DMA,
        ],
        compiler_params=pltpu.CompilerParams(
            use_tc_tiling_on_sc=False, needs_layout_passes=False
        ),
    )(xf, table)
    return jnp.transpose(out, (2, 4, 0, 1, 3)).reshape(16384, 50, DIM)
